# baseline (speedup 1.0000x reference)
import functools

import jax
import jax.numpy as jnp
from jax.experimental import pallas as pl
from jax.experimental.pallas import tpu as pltpu
from jax.experimental.pallas import tpu_sc as plsc

_E = 8
_K = 2
_T = 4096
_D = 768
_H = _D // 2
_CAP = int(_T * _K / _E * 1.25)
_NSLOT = _E * _CAP
_NPAIR = _T * _K

_BT_R = 256
_BC = 256
_BT_C = 512

_NW = 32
_PPW = _NPAIR // _NW
_CH = 128


def _router_body(x_ref, wg_ref, dst_ref, src_ref, cpair_ref, carry_ref):
    b = pl.program_id(0)

    @pl.when(b == 0)
    def _():
        carry_ref[...] = jnp.zeros_like(carry_ref)

    x = x_ref[...]
    logits = jnp.dot(x, wg_ref[...], preferred_element_type=jnp.float32)

    iota_e = jax.lax.broadcasted_iota(jnp.int32, logits.shape, 1)
    m1 = jnp.max(logits, axis=1, keepdims=True)
    i1 = jnp.min(jnp.where(logits == m1, iota_e, _E), axis=1, keepdims=True)
    sel1 = iota_e == i1
    l2 = jnp.where(sel1, -jnp.inf, logits)
    m2 = jnp.max(l2, axis=1, keepdims=True)
    i2 = jnp.min(jnp.where(l2 == m2, iota_e, _E), axis=1, keepdims=True)
    sel2 = iota_e == i2

    g1 = 1.0 / (1.0 + jnp.exp(m2 - m1))
    g2 = 1.0 - g1

    cnt = sel1.astype(jnp.float32) + sel2.astype(jnp.float32)
    ii = jax.lax.broadcasted_iota(jnp.int32, (_BT_R, _BT_R), 0)
    jj = jax.lax.broadcasted_iota(jnp.int32, (_BT_R, _BT_R), 1)
    lt = (jj < ii).astype(jnp.float32)
    cum = jnp.dot(lt, cnt, preferred_element_type=jnp.float32) + carry_ref[...]
    carry_ref[...] += jnp.sum(cnt, axis=0, keepdims=True)

    pos1 = jnp.sum(jnp.where(sel1, cum, 0.0), axis=1, keepdims=True).astype(jnp.int32)
    pos2 = jnp.sum(jnp.where(sel2, cum, 0.0), axis=1, keepdims=True).astype(jnp.int32)
    keep1 = pos1 < _CAP
    keep2 = pos2 < _CAP

    slot1 = i1 * _CAP + pos1
    slot2 = i2 * _CAP + pos2
    dst1 = jnp.where(keep1, slot1, _NSLOT)
    dst2 = jnp.where(keep2, slot2, _NSLOT)
    src1 = jnp.where(keep1, slot1, i1 * _CAP)
    src2 = jnp.where(keep2, slot2, i2 * _CAP)
    c1 = g1 * keep1.astype(jnp.float32)
    c2 = g2 * keep2.astype(jnp.float32)

    dst_ref[...] = jnp.concatenate([dst1, dst2], axis=1)
    src_ref[...] = jnp.concatenate([src1, src2], axis=1)
    cpair_ref[...] = jnp.concatenate([c1, c2], axis=1)


def _expert_body(buf_ref, w1_ref, w2_ref, y_ref):
    h = jnp.maximum(
        jnp.dot(buf_ref[...], w1_ref[0], preferred_element_type=jnp.float32), 0.0)
    y_ref[...] = jnp.maximum(
        jnp.dot(h, w2_ref[0], preferred_element_type=jnp.float32), 0.0)


def _combine_body(x_ref, y0_ref, y1_ref, cpair_ref, out_ref):
    c = cpair_ref[...]
    out_ref[...] = (x_ref[...]
                    + c[:, 0:1] * y0_ref[...]
                    + c[:, 1:2] * y1_ref[...])


def _dispatch_sc(x_hbm, tok_hbm, dst_hbm, buf_hbm, tokv, dstv, rows, sem1, sem2):
    wid = jax.lax.axis_index("s") * 2 + jax.lax.axis_index("c")
    base = wid * _PPW
    for ci in range(_PPW // _CH):
        off = base + ci * _CH
        pltpu.sync_copy(tok_hbm.at[pl.ds(off, _CH)], tokv)
        pltpu.sync_copy(dst_hbm.at[pl.ds(off, _CH)], dstv)
        pltpu.async_copy(x_hbm.at[tokv], rows, sem1).wait()
        pltpu.async_copy(rows, buf_hbm.at[dstv], sem2).wait()


def _gather_sc(y_hbm, src_hbm, yp_hbm, srcv, rows, sem1):
    wid = jax.lax.axis_index("s") * 2 + jax.lax.axis_index("c")
    base = wid * _PPW
    for ci in range(_PPW // _CH):
        off = base + ci * _CH
        pltpu.sync_copy(src_hbm.at[pl.ds(off, _CH)], srcv)
        pltpu.async_copy(y_hbm.at[srcv], rows, sem1).wait()
        pltpu.sync_copy(rows, yp_hbm.at[pl.ds(off, _CH)])


def kernel(x, Wg, W1, W2):
    dst, src, cpair = pl.pallas_call(
        _router_body,
        grid=(_T // _BT_R,),
        in_specs=[
            pl.BlockSpec((_BT_R, _D), lambda b: (b, 0)),
            pl.BlockSpec((_D, _E), lambda b: (0, 0)),
        ],
        out_specs=[
            pl.BlockSpec((_BT_R, _K), lambda b: (b, 0)),
            pl.BlockSpec((_BT_R, _K), lambda b: (b, 0)),
            pl.BlockSpec((_BT_R, _K), lambda b: (b, 0)),
        ],
        out_shape=[
            jax.ShapeDtypeStruct((_T, _K), jnp.int32),
            jax.ShapeDtypeStruct((_T, _K), jnp.int32),
            jax.ShapeDtypeStruct((_T, _K), jnp.float32),
        ],
        scratch_shapes=[pltpu.VMEM((1, _E), jnp.float32)],
    )(x, Wg)

    tok = jnp.repeat(jnp.arange(_T, dtype=jnp.int32), _K)
    dst_flat = dst.reshape(_NPAIR)
    src_km = src.T.reshape(_NPAIR)

    mesh = plsc.VectorSubcoreMesh(core_axis_name="c", subcore_axis_name="s")

    dispatch = functools.partial(
        pl.kernel,
        mesh=mesh,
        out_type=jax.ShapeDtypeStruct((_NSLOT + 8, _D), jnp.float32),
        scratch_types=[
            pltpu.VMEM((_CH,), jnp.int32),
            pltpu.VMEM((_CH,), jnp.int32),
            pltpu.VMEM((_CH, _D), jnp.float32),
            pltpu.SemaphoreType.DMA,
            pltpu.SemaphoreType.DMA,
        ],
    )(_dispatch_sc)
    buf = dispatch(x, tok, dst_flat)

    y = pl.pallas_call(
        _expert_body,
        grid=(_E, _CAP // _BC),
        in_specs=[
            pl.BlockSpec((_BC, _D), lambda e, cb: (e * (_CAP // _BC) + cb, 0)),
            pl.BlockSpec((1, _D, _H), lambda e, cb: (e, 0, 0)),
            pl.BlockSpec((1, _H, _D), lambda e, cb: (e, 0, 0)),
        ],
        out_specs=pl.BlockSpec((_BC, _D), lambda e, cb: (e * (_CAP // _BC) + cb, 0)),
        out_shape=jax.ShapeDtypeStruct((_NSLOT, _D), jnp.float32),
    )(buf, W1, W2)

    gather = functools.partial(
        pl.kernel,
        mesh=mesh,
        out_type=jax.ShapeDtypeStruct((_NPAIR, _D), jnp.float32),
        scratch_types=[
            pltpu.VMEM((_CH,), jnp.int32),
            pltpu.VMEM((_CH, _D), jnp.float32),
            pltpu.SemaphoreType.DMA,
        ],
    )(_gather_sc)
    yp = gather(y, src_km)

    out = pl.pallas_call(
        _combine_body,
        grid=(_T // _BT_C,),
        in_specs=[
            pl.BlockSpec((_BT_C, _D), lambda b: (b, 0)),
            pl.BlockSpec((_BT_C, _D), lambda b: (b, 0)),
            pl.BlockSpec((_BT_C, _D), lambda b: (_T // _BT_C + b, 0)),
            pl.BlockSpec((_BT_C, _K), lambda b: (b, 0)),
        ],
        out_specs=pl.BlockSpec((_BT_C, _D), lambda b: (b, 0)),
        out_shape=jax.ShapeDtypeStruct((_T, _D), jnp.float32),
    )(x, yp, yp, cpair)
    return out
